# baseline (speedup 1.0000x reference)
import functools

import jax
import jax.numpy as jnp
from jax import lax
from jax.experimental import pallas as pl
from jax.experimental.pallas import tpu as pltpu
from jax.experimental.pallas import tpu_sc as plsc

NPL = 50000
NPOS = 5000
NTEAM = 320
FEAT = 128
HID = 128
EMB = 64
E_CHEM = 800000
E_P2P = 800000
E_P2T = 80000

NC, NS, L = 2, 16, 16
NW = NC * NS
CH = 128
W128 = 128

POSR = 5120
TEAMR = 384

NCH_CHEM = 196
NCH_P2T = 20
E_CHEM_PAD = NW * CH * NCH_CHEM
E_P2P_PAD = NW * CH * NCH_CHEM
E_P2T_PAD = NW * CH * NCH_P2T

_MESH = plsc.VectorSubcoreMesh(core_axis_name="c", subcore_axis_name="s")


def _enc_body(pf, w1, b1, w2, b2, wq, bq, wk, bk, wv, bv,
              pe_o, t1_o, t2_o):
    h = jnp.maximum(pf[...] @ w1[...] + b1[...], 0.0)
    pe = h @ w2[...] + b2[...]
    pe_o[...] = pe
    q = pe @ wq[...] + bq[...]
    k = pe @ wk[...] + bk[...]
    v = pe @ wv[...] + bv[...]
    t1_o[...] = jnp.concatenate(
        [q, jnp.zeros(q.shape, jnp.float32)], axis=1)
    t2_o[...] = jnp.concatenate([k, v], axis=1)


def _encode_qkv(pf, w1, b1, w2, b2, wq, bq, wk, bk, wv, bv):
    blk = 1000
    grid = NPL // blk
    full = lambda a: pl.BlockSpec(a.shape, lambda i: (0,) * a.ndim)
    return pl.pallas_call(
        _enc_body,
        grid=(grid,),
        in_specs=[pl.BlockSpec((blk, FEAT), lambda i: (i, 0)),
                  full(w1), full(b1), full(w2), full(b2),
                  full(wq), full(bq), full(wk), full(bk), full(wv), full(bv)],
        out_specs=[pl.BlockSpec((blk, EMB), lambda i: (i, 0)),
                   pl.BlockSpec((blk, W128), lambda i: (i, 0)),
                   pl.BlockSpec((blk, W128), lambda i: (i, 0))],
        out_shape=[jax.ShapeDtypeStruct((NPL, EMB), jnp.float32),
                   jax.ShapeDtypeStruct((NPL, W128), jnp.float32),
                   jax.ShapeDtypeStruct((NPL, W128), jnp.float32)],
    )(pf, w1, b1, w2, b2, wq, bq, wk, bk, wv, bv)


_CHEM_EPW = E_CHEM_PAD // NW


def _gather2_body(t1, t2, srcp, dstp,
                  qg, kvg,
                  sidx0, sidx1, didx0, didx1, qb0, qb1, kvb0, kvb1,
                  si0, si1, sg0, sg1, sw0, sw1):
    cid = lax.axis_index("c")
    sid = lax.axis_index("s")
    wid = sid * NC + cid
    ebase = wid * _CHEM_EPW
    last = NCH_CHEM - 1

    def idx(c, sx, dx, si):
        off = pl.multiple_of(ebase + c * CH, CH)
        pltpu.async_copy(srcp.at[pl.ds(off, CH)], sx, si)
        pltpu.async_copy(dstp.at[pl.ds(off, CH)], dx, si)

    def iwait(sx, dx, si):
        pltpu.make_async_copy(srcp.at[pl.ds(0, CH)], sx, si).wait()
        pltpu.make_async_copy(dstp.at[pl.ds(0, CH)], dx, si).wait()

    def g(sx, dx, qb, kvb, sg):
        pltpu.async_copy(t1.at[dx], qb, sg)
        pltpu.async_copy(t2.at[sx], kvb, sg)

    def gwait(qb, kvb, sg):
        pltpu.make_async_copy(t1.at[pl.ds(0, CH)], qb, sg).wait()
        pltpu.make_async_copy(t2.at[pl.ds(0, CH)], kvb, sg).wait()

    def w(c, qb, kvb, sw):
        off = pl.multiple_of(ebase + c * CH, CH)
        pltpu.async_copy(qb, qg.at[pl.ds(off, CH)], sw)
        pltpu.async_copy(kvb, kvg.at[pl.ds(off, CH)], sw)

    def wwait(qb, kvb, sw):
        pltpu.make_async_copy(qb, qg.at[pl.ds(0, CH)], sw).wait()
        pltpu.make_async_copy(kvb, kvg.at[pl.ds(0, CH)], sw).wait()

    idx(0, sidx0, didx0, si0)
    idx(1, sidx1, didx1, si1)
    iwait(sidx0, didx0, si0)
    g(sidx0, didx0, qb0, kvb0, sg0)

    def pair(j, _):
        c0 = j * 2
        c1 = c0 + 1
        iwait(sidx1, didx1, si1)
        g(sidx1, didx1, qb1, kvb1, sg1)
        gwait(qb0, kvb0, sg0)
        w(c0, qb0, kvb0, sw0)
        wwait(qb0, kvb0, sw0)
        idx(jnp.minimum(c0 + 2, last), sidx0, didx0, si0)
        iwait(sidx0, didx0, si0)
        g(sidx0, didx0, qb0, kvb0, sg0)
        gwait(qb1, kvb1, sg1)
        w(c1, qb1, kvb1, sw1)
        wwait(qb1, kvb1, sw1)
        idx(jnp.minimum(c1 + 2, last), sidx1, didx1, si1)
        return 0

    lax.fori_loop(0, NCH_CHEM // 2, pair, 0)
    gwait(qb0, kvb0, sg0)
    iwait(sidx1, didx1, si1)


def _gather2(t1, t2, srcp, dstp):
    eg = jax.ShapeDtypeStruct((E_CHEM_PAD, W128), jnp.float32)
    f = pl.kernel(
        _gather2_body,
        out_type=(eg, eg),
        mesh=_MESH,
        scratch_types=[
            pltpu.VMEM((CH,), jnp.int32),
            pltpu.VMEM((CH,), jnp.int32),
            pltpu.VMEM((CH,), jnp.int32),
            pltpu.VMEM((CH,), jnp.int32),
            pltpu.VMEM((CH, W128), jnp.float32),
            pltpu.VMEM((CH, W128), jnp.float32),
            pltpu.VMEM((CH, W128), jnp.float32),
            pltpu.VMEM((CH, W128), jnp.float32),
            pltpu.SemaphoreType.DMA,
            pltpu.SemaphoreType.DMA,
            pltpu.SemaphoreType.DMA,
            pltpu.SemaphoreType.DMA,
            pltpu.SemaphoreType.DMA,
            pltpu.SemaphoreType.DMA,
        ],
    )
    return f(t1, t2, srcp, dstp)


_WB = 2048
_WGRID = E_CHEM_PAD // _WB


def _wmsg_body(qg, kvg, ones64, m_o, z_o):
    i = pl.program_id(0)
    inv = 1.0 / (EMB ** 0.5)
    q = qg[...][:, :EMB]
    kv = kvg[...]
    k = kv[:, :EMB]
    v = kv[:, EMB:]
    s = ((q * k) @ ones64[...]) * inv
    row = lax.broadcasted_iota(jnp.int32, (_WB, EMB), 0) + i * _WB
    w = jnp.where(row < E_CHEM, jnp.exp(s), 0.0)
    m_o[...] = w * v
    zb = jnp.sum(w) * (1.0 / EMB)

    @pl.when(i == 0)
    def _():
        z_o[...] = jnp.zeros((8, 128), jnp.float32)

    z_o[...] += jnp.full((8, 128), zb, jnp.float32)


def _wmsg(qg, kvg):
    ones64 = jnp.ones((EMB, EMB), jnp.float32)
    return pl.pallas_call(
        _wmsg_body,
        grid=(_WGRID,),
        in_specs=[pl.BlockSpec((_WB, W128), lambda i: (i, 0)),
                  pl.BlockSpec((_WB, W128), lambda i: (i, 0)),
                  pl.BlockSpec((EMB, EMB), lambda i: (0, 0))],
        out_specs=[pl.BlockSpec((_WB, EMB), lambda i: (i, 0)),
                   pl.BlockSpec((8, 128), lambda i: (0, 0))],
        out_shape=[jax.ShapeDtypeStruct((E_CHEM_PAD, EMB), jnp.float32),
                   jax.ShapeDtypeStruct((8, 128), jnp.float32)],
    )(qg, kvg, ones64)


def _scatadd_body(msgs, dstp, zinit, out,
                  didx0, didx1, rb0, rb1, aggsh,
                  si0, si1, sg0, sg1, *, nch, epw, nrows, width):
    cid = lax.axis_index("c")
    sid = lax.axis_index("s")
    wid = sid * NC + cid
    ebase = wid * epw
    last = nch - 1

    @pl.when(sid == 0)
    def _():
        pltpu.sync_copy(zinit, aggsh)
    plsc.subcore_barrier()

    def idx(c, dx, si):
        off = pl.multiple_of(ebase + c * CH, CH)
        pltpu.async_copy(dstp.at[pl.ds(off, CH)], dx, si)

    def iwait(dx, si):
        pltpu.make_async_copy(dstp.at[pl.ds(0, CH)], dx, si).wait()

    def g(c, rb, sg):
        off = pl.multiple_of(ebase + c * CH, CH)
        pltpu.async_copy(msgs.at[pl.ds(off, CH)], rb, sg)

    def gwait(rb, sg):
        pltpu.make_async_copy(msgs.at[pl.ds(0, CH)], rb, sg).wait()

    idx(0, didx0, si0)
    idx(1, didx1, si1)
    g(0, rb0, sg0)
    g(1, rb1, sg1)

    def pair(j, _):
        c0 = j * 2
        c1 = c0 + 1
        gwait(rb0, sg0)
        iwait(didx0, si0)
        pltpu.sync_copy(rb0, aggsh.at[didx0], add=True)
        idx(jnp.minimum(c0 + 2, last), didx0, si0)
        g(jnp.minimum(c0 + 2, last), rb0, sg0)
        gwait(rb1, sg1)
        iwait(didx1, si1)
        pltpu.sync_copy(rb1, aggsh.at[didx1], add=True)
        idx(jnp.minimum(c1 + 2, last), didx1, si1)
        g(jnp.minimum(c1 + 2, last), rb1, sg1)
        return 0

    lax.fori_loop(0, nch // 2, pair, 0)
    gwait(rb0, sg0)
    gwait(rb1, sg1)
    iwait(didx0, si0)
    iwait(didx1, si1)
    plsc.subcore_barrier()
    rp = nrows // NS
    pltpu.sync_copy(aggsh.at[pl.ds(sid * rp, rp)],
                    out.at[cid, pl.ds(sid * rp, rp)])


def _scatadd(msgs, dstp, nrows, e_pad, width, nch):
    epw = e_pad // NW
    zinit = jnp.zeros((nrows, width), jnp.float32)
    body = functools.partial(_scatadd_body, nch=nch, epw=epw,
                             nrows=nrows, width=width)
    f = pl.kernel(
        body,
        out_type=jax.ShapeDtypeStruct((NC, nrows, width), jnp.float32),
        mesh=_MESH,
        scratch_types=[
            pltpu.VMEM((CH,), jnp.int32),
            pltpu.VMEM((CH,), jnp.int32),
            pltpu.VMEM((CH, width), jnp.float32),
            pltpu.VMEM((CH, width), jnp.float32),
            pltpu.VMEM_SHARED((nrows, width), jnp.float32),
            pltpu.SemaphoreType.DMA,
            pltpu.SemaphoreType.DMA,
            pltpu.SemaphoreType.DMA,
            pltpu.SemaphoreType.DMA,
        ],
    )
    return f(msgs, dstp, zinit)


def _agg_body(table, srcp, dstp, zinit, out,
              sidx0, sidx1, didx0, didx1, rb0, rb1,
              aggsh, si0, si1, sg0, sg1, *, nch, epw, nrows):
    cid = lax.axis_index("c")
    sid = lax.axis_index("s")
    wid = sid * NC + cid
    ebase = wid * epw
    last = nch - 1

    @pl.when(sid == 0)
    def _():
        pltpu.sync_copy(zinit, aggsh)
    plsc.subcore_barrier()

    def idx(c, sx, dx, si):
        off = pl.multiple_of(ebase + c * CH, CH)
        pltpu.async_copy(srcp.at[pl.ds(off, CH)], sx, si)
        pltpu.async_copy(dstp.at[pl.ds(off, CH)], dx, si)

    def iwait(sx, dx, si):
        pltpu.make_async_copy(srcp.at[pl.ds(0, CH)], sx, si).wait()
        pltpu.make_async_copy(dstp.at[pl.ds(0, CH)], dx, si).wait()

    def g(sx, rb, sg):
        pltpu.async_copy(table.at[sx], rb, sg)

    def gwait(rb, sg):
        pltpu.make_async_copy(table.at[pl.ds(0, CH)], rb, sg).wait()

    idx(0, sidx0, didx0, si0)
    idx(1, sidx1, didx1, si1)
    iwait(sidx0, didx0, si0)
    g(sidx0, rb0, sg0)

    def pair(j, _):
        c0 = j * 2
        c1 = c0 + 1
        iwait(sidx1, didx1, si1)
        g(sidx1, rb1, sg1)
        gwait(rb0, sg0)
        pltpu.sync_copy(rb0, aggsh.at[didx0], add=True)
        idx(jnp.minimum(c0 + 2, last), sidx0, didx0, si0)
        iwait(sidx0, didx0, si0)
        g(sidx0, rb0, sg0)
        gwait(rb1, sg1)
        pltpu.sync_copy(rb1, aggsh.at[didx1], add=True)
        idx(jnp.minimum(c1 + 2, last), sidx1, didx1, si1)
        return 0

    lax.fori_loop(0, nch // 2, pair, 0)
    gwait(rb0, sg0)
    iwait(sidx1, didx1, si1)
    plsc.subcore_barrier()
    rp = nrows // NS
    pltpu.sync_copy(aggsh.at[pl.ds(sid * rp, rp)],
                    out.at[cid, pl.ds(sid * rp, rp)])


def _agg_pass(table, srcp, dstp, nrows, e_pad, nch):
    epw = e_pad // NW
    zinit = jnp.zeros((nrows, W128), jnp.float32)
    body = functools.partial(_agg_body, nch=nch, epw=epw, nrows=nrows)
    f = pl.kernel(
        body,
        out_type=jax.ShapeDtypeStruct((NC, nrows, W128), jnp.float32),
        mesh=_MESH,
        scratch_types=[
            pltpu.VMEM((CH,), jnp.int32),
            pltpu.VMEM((CH,), jnp.int32),
            pltpu.VMEM((CH,), jnp.int32),
            pltpu.VMEM((CH,), jnp.int32),
            pltpu.VMEM((CH, W128), jnp.float32),
            pltpu.VMEM((CH, W128), jnp.float32),
            pltpu.VMEM_SHARED((nrows, W128), jnp.float32),
            pltpu.SemaphoreType.DMA,
            pltpu.SemaphoreType.DMA,
            pltpu.SemaphoreType.DMA,
            pltpu.SemaphoreType.DMA,
        ],
    )
    return f(table, srcp, dstp, zinit)


def _msg1_body(pe5, p0, p1, zp, mw, mb, out):
    z = jnp.max(zp[...])
    agg = (p0[...][:NPOS] + p1[...][:NPOS]) / z
    pen = pe5[...] + agg
    m = jnp.maximum(pen @ mw[...] + mb[...], 0.0)
    top = jnp.concatenate(
        [m, jnp.ones((NPOS, 16), jnp.float32),
         jnp.zeros((NPOS, W128 - EMB - 16), jnp.float32)], axis=1)
    out[...] = jnp.concatenate(
        [top, jnp.zeros((POSR - NPOS, W128), jnp.float32)], axis=0)


def _msg1(pe5, p0, p1, zp, mw, mb):
    return pl.pallas_call(
        _msg1_body,
        out_shape=jax.ShapeDtypeStruct((POSR, W128), jnp.float32),
    )(pe5, p0, p1, zp, mw, mb)


def _msg2_body(p0, p1, idx, ptab, gsel, esel, u1, u2, ub, tw, tb, out):
    s5 = (p0[...] + p1[...])[:NPOS]
    cnt = jnp.maximum(s5 @ esel[...], 1.0)
    agg = (s5 @ gsel[...]) / cnt
    c2 = agg @ u2[...] + ub[...]
    oh = (idx[...] == lax.broadcasted_iota(jnp.int32, (NPOS, 16), 1)
          ).astype(jnp.float32)
    pos = oh @ ptab[...]
    for _ in range(3):
        pos = jnp.maximum(pos @ u1[...] + c2, 0.0)
    m = jnp.maximum(pos[:NTEAM] @ tw[...] + tb[...], 0.0)
    top = jnp.concatenate(
        [m, jnp.ones((NTEAM, 16), jnp.float32),
         jnp.zeros((NTEAM, W128 - EMB - 16), jnp.float32)], axis=1)
    out[...] = jnp.concatenate(
        [top, jnp.zeros((TEAMR - NTEAM, W128), jnp.float32)], axis=0)


def _msg2(p0, p1, idx, ptab, gsel, esel, u1, u2, ub, tw, tb):
    return pl.pallas_call(
        _msg2_body,
        out_shape=jax.ShapeDtypeStruct((TEAMR, W128), jnp.float32),
    )(p0, p1, idx, ptab, gsel, esel, u1, u2, ub, tw, tb)


def _team_body(p0, p1, idx, ttab, gsel, esel, u1, u2, ub, out):
    s = (p0[...] + p1[...])[:NTEAM]
    cnt = jnp.maximum(s @ esel[...], 1.0)
    agg = (s @ gsel[...]) / cnt
    c2 = agg @ u2[...] + ub[...]
    oh = (idx[...] == lax.broadcasted_iota(jnp.int32, (NTEAM, 32), 1)
          ).astype(jnp.float32)
    team = oh @ ttab[...]
    for _ in range(3):
        team = jnp.maximum(team @ u1[...] + c2, 0.0)
    out[...] = team


def _team(p0, p1, idx, ttab, gsel, esel, u1, u2, ub):
    return pl.pallas_call(
        _team_body,
        out_shape=jax.ShapeDtypeStruct((NTEAM, EMB), jnp.float32),
    )(p0, p1, idx, ttab, gsel, esel, u1, u2, ub)


def _pad_edges(arr, n_pad, fill):
    return jnp.concatenate(
        [arr.astype(jnp.int32),
         jnp.full((n_pad - arr.shape[0],), fill, jnp.int32)])


@jax.jit
def kernel(player_features, position_indices, team_indices,
           player_to_position_edges, position_to_team_edges, chemistry_edges,
           enc_W1, enc_b1, enc_W2, enc_b2,
           attn_Wq, attn_bq, attn_Wk, attn_bk, attn_Wv, attn_bv,
           pos_table, team_table,
           p2p_msg_W, p2p_msg_b, p2p_upd_W, p2p_upd_b,
           p2t_msg_W, p2t_msg_b, p2t_upd_W, p2t_upd_b):
    r2 = lambda b: b.reshape(1, -1)

    pe, t1, t2 = _encode_qkv(
        player_features, enc_W1, r2(enc_b1), enc_W2, r2(enc_b2),
        attn_Wq, r2(attn_bq), attn_Wk, r2(attn_bk), attn_Wv, r2(attn_bv))

    csrc = _pad_edges(chemistry_edges[0], E_CHEM_PAD, 0)
    cdst = _pad_edges(chemistry_edges[1], E_CHEM_PAD, 0)
    cdst_cl = jnp.minimum(cdst, NPOS)
    qg, kvg = _gather2(t1, t2, csrc, cdst)
    msgs, zp = _wmsg(qg, kvg)
    aggc = _scatadd(msgs, cdst_cl, POSR, E_CHEM_PAD, EMB, NCH_CHEM)

    mp = _msg1(pe[:NPOS], aggc[0], aggc[1], zp,
               p2p_msg_W, r2(p2p_msg_b))

    psrc = _pad_edges(player_to_position_edges[0], E_P2P_PAD, 0)
    pdst = _pad_edges(player_to_position_edges[1], E_P2P_PAD, NPOS)
    aggp = _agg_pass(mp, psrc, pdst, POSR, E_P2P_PAD, NCH_CHEM)

    gsel = jnp.concatenate(
        [jnp.eye(EMB, dtype=jnp.float32),
         jnp.zeros((W128 - EMB, EMB), jnp.float32)], axis=0)
    esel = jnp.concatenate(
        [jnp.zeros((EMB, EMB), jnp.float32),
         jnp.full((16, EMB), 1.0 / 16.0, jnp.float32),
         jnp.zeros((W128 - EMB - 16, EMB), jnp.float32)], axis=0)

    ptab16 = jnp.concatenate(
        [pos_table, jnp.zeros((6, EMB), jnp.float32)], axis=0)
    mt = _msg2(aggp[0], aggp[1], position_indices.astype(jnp.int32)[:, None],
               ptab16, gsel, esel,
               p2p_upd_W[:EMB], p2p_upd_W[EMB:], r2(p2p_upd_b),
               p2t_msg_W, r2(p2t_msg_b))

    tsrc = _pad_edges(position_to_team_edges[0], E_P2T_PAD, 0)
    tdst = _pad_edges(position_to_team_edges[1], E_P2T_PAD, NTEAM)
    aggt = _agg_pass(mt, tsrc, tdst, TEAMR, E_P2T_PAD, NCH_P2T)

    return _team(aggt[0], aggt[1], team_indices.astype(jnp.int32)[:, None],
                 team_table, gsel, esel,
                 p2t_upd_W[:EMB], p2t_upd_W[EMB:], r2(p2t_upd_b))


# speedup vs baseline: 1.0824x; 1.0824x over previous
import functools

import jax
import jax.numpy as jnp
from jax import lax
from jax.experimental import pallas as pl
from jax.experimental.pallas import tpu as pltpu
from jax.experimental.pallas import tpu_sc as plsc

NPL = 50000
NPOS = 5000
NTEAM = 320
FEAT = 128
HID = 128
EMB = 64
E_CHEM = 800000
E_P2P = 800000
E_P2T = 80000

NC, NS, L = 2, 16, 16
NW = NC * NS
CH = 128
W128 = 128

POSR = 5120
TEAMR = 384

NCH_CHEM = 196
NCH_P2T = 20
E_CHEM_PAD = NW * CH * NCH_CHEM
E_P2P_PAD = NW * CH * NCH_CHEM
E_P2T_PAD = NW * CH * NCH_P2T

_MESH = plsc.VectorSubcoreMesh(core_axis_name="c", subcore_axis_name="s")


def _enc_body(pf, w1, b1, w2, b2, wq, bq, wk, bk, wv, bv,
              pe_o, t1_o, t2_o):
    h = jnp.maximum(pf[...] @ w1[...] + b1[...], 0.0)
    pe = h @ w2[...] + b2[...]
    pe_o[...] = pe
    q = pe @ wq[...] + bq[...]
    k = pe @ wk[...] + bk[...]
    v = pe @ wv[...] + bv[...]
    t1_o[...] = jnp.concatenate(
        [q, jnp.zeros(q.shape, jnp.float32)], axis=1)
    t2_o[...] = jnp.concatenate([k, v], axis=1)


def _encode_qkv(pf, w1, b1, w2, b2, wq, bq, wk, bk, wv, bv):
    blk = 1000
    grid = NPL // blk
    full = lambda a: pl.BlockSpec(a.shape, lambda i: (0,) * a.ndim)
    return pl.pallas_call(
        _enc_body,
        grid=(grid,),
        in_specs=[pl.BlockSpec((blk, FEAT), lambda i: (i, 0)),
                  full(w1), full(b1), full(w2), full(b2),
                  full(wq), full(bq), full(wk), full(bk), full(wv), full(bv)],
        out_specs=[pl.BlockSpec((blk, EMB), lambda i: (i, 0)),
                   pl.BlockSpec((blk, W128), lambda i: (i, 0)),
                   pl.BlockSpec((blk, W128), lambda i: (i, 0))],
        out_shape=[jax.ShapeDtypeStruct((NPL, EMB), jnp.float32),
                   jax.ShapeDtypeStruct((NPL, W128), jnp.float32),
                   jax.ShapeDtypeStruct((NPL, W128), jnp.float32)],
    )(pf, w1, b1, w2, b2, wq, bq, wk, bk, wv, bv)


_CHEM_EPW = E_CHEM_PAD // NW


def _gather2_body(t1, t2, srcp, dstp,
                  qg, kvg,
                  sidx0, sidx1, didx0, didx1, qb0, qb1, kvb0, kvb1,
                  si0, si1, sg0, sg1, sw0, sw1):
    cid = lax.axis_index("c")
    sid = lax.axis_index("s")
    wid = sid * NC + cid
    ebase = wid * _CHEM_EPW
    last = NCH_CHEM - 1

    def idx(c, sx, dx, si):
        off = pl.multiple_of(ebase + c * CH, CH)
        pltpu.async_copy(srcp.at[pl.ds(off, CH)], sx, si)
        pltpu.async_copy(dstp.at[pl.ds(off, CH)], dx, si)

    def iwait(sx, dx, si):
        pltpu.make_async_copy(srcp.at[pl.ds(0, CH)], sx, si).wait()
        pltpu.make_async_copy(dstp.at[pl.ds(0, CH)], dx, si).wait()

    def g(sx, dx, qb, kvb, sg):
        pltpu.async_copy(t1.at[dx], qb, sg)
        pltpu.async_copy(t2.at[sx], kvb, sg)

    def gwait(qb, kvb, sg):
        pltpu.make_async_copy(t1.at[pl.ds(0, CH)], qb, sg).wait()
        pltpu.make_async_copy(t2.at[pl.ds(0, CH)], kvb, sg).wait()

    def w(c, qb, kvb, sw):
        off = pl.multiple_of(ebase + c * CH, CH)
        pltpu.async_copy(qb, qg.at[pl.ds(off, CH)], sw)
        pltpu.async_copy(kvb, kvg.at[pl.ds(off, CH)], sw)

    def wwait(qb, kvb, sw):
        pltpu.make_async_copy(qb, qg.at[pl.ds(0, CH)], sw).wait()
        pltpu.make_async_copy(kvb, kvg.at[pl.ds(0, CH)], sw).wait()

    idx(0, sidx0, didx0, si0)
    idx(1, sidx1, didx1, si1)
    iwait(sidx0, didx0, si0)
    g(sidx0, didx0, qb0, kvb0, sg0)

    def pair(j, _):
        c0 = j * 2
        c1 = c0 + 1
        iwait(sidx1, didx1, si1)
        g(sidx1, didx1, qb1, kvb1, sg1)
        gwait(qb0, kvb0, sg0)
        w(c0, qb0, kvb0, sw0)
        wwait(qb0, kvb0, sw0)
        idx(jnp.minimum(c0 + 2, last), sidx0, didx0, si0)
        iwait(sidx0, didx0, si0)
        g(sidx0, didx0, qb0, kvb0, sg0)
        gwait(qb1, kvb1, sg1)
        w(c1, qb1, kvb1, sw1)
        wwait(qb1, kvb1, sw1)
        idx(jnp.minimum(c1 + 2, last), sidx1, didx1, si1)
        return 0

    lax.fori_loop(0, NCH_CHEM // 2, pair, 0)
    gwait(qb0, kvb0, sg0)
    iwait(sidx1, didx1, si1)


def _gather2(t1, t2, srcp, dstp):
    eg = jax.ShapeDtypeStruct((E_CHEM_PAD, W128), jnp.float32)
    f = pl.kernel(
        _gather2_body,
        out_type=(eg, eg),
        mesh=_MESH,
        scratch_types=[
            pltpu.VMEM((CH,), jnp.int32),
            pltpu.VMEM((CH,), jnp.int32),
            pltpu.VMEM((CH,), jnp.int32),
            pltpu.VMEM((CH,), jnp.int32),
            pltpu.VMEM((CH, W128), jnp.float32),
            pltpu.VMEM((CH, W128), jnp.float32),
            pltpu.VMEM((CH, W128), jnp.float32),
            pltpu.VMEM((CH, W128), jnp.float32),
            pltpu.SemaphoreType.DMA,
            pltpu.SemaphoreType.DMA,
            pltpu.SemaphoreType.DMA,
            pltpu.SemaphoreType.DMA,
            pltpu.SemaphoreType.DMA,
            pltpu.SemaphoreType.DMA,
        ],
    )
    return f(t1, t2, srcp, dstp)


_WB = 2048
_WGRID = E_CHEM_PAD // _WB


def _wmsg_body(qg, kvg, ones64, m_o, z_o):
    i = pl.program_id(0)
    inv = 1.0 / (EMB ** 0.5)
    q = qg[...][:, :EMB]
    kv = kvg[...]
    k = kv[:, :EMB]
    v = kv[:, EMB:]
    s = ((q * k) @ ones64[...]) * inv
    row = lax.broadcasted_iota(jnp.int32, (_WB, EMB), 0) + i * _WB
    w = jnp.where(row < E_CHEM, jnp.exp(s), 0.0)
    m_o[...] = w * v
    zb = jnp.sum(w) * (1.0 / EMB)

    @pl.when(i == 0)
    def _():
        z_o[...] = jnp.zeros((8, 128), jnp.float32)

    z_o[...] += jnp.full((8, 128), zb, jnp.float32)


def _wmsg(qg, kvg):
    ones64 = jnp.ones((EMB, EMB), jnp.float32)
    return pl.pallas_call(
        _wmsg_body,
        grid=(_WGRID,),
        in_specs=[pl.BlockSpec((_WB, W128), lambda i: (i, 0)),
                  pl.BlockSpec((_WB, W128), lambda i: (i, 0)),
                  pl.BlockSpec((EMB, EMB), lambda i: (0, 0))],
        out_specs=[pl.BlockSpec((_WB, EMB), lambda i: (i, 0)),
                   pl.BlockSpec((8, 128), lambda i: (0, 0))],
        out_shape=[jax.ShapeDtypeStruct((E_CHEM_PAD, EMB), jnp.float32),
                   jax.ShapeDtypeStruct((8, 128), jnp.float32)],
    )(qg, kvg, ones64)


def _scatadd_body(msgs, dstp3, zinit, out, didxa, rb0, rb1, aggsh,
                  sg0, sg1, *, nch, epw, nrows, width):
    cid = lax.axis_index("c")
    sid = lax.axis_index("s")
    wid = sid * NC + cid
    ebase = wid * epw

    @pl.when(sid == 0)
    def _():
        pltpu.sync_copy(zinit, aggsh)
    pltpu.sync_copy(dstp3.at[wid], didxa)
    plsc.subcore_barrier()
    last = nch - 1

    def g(c, rb, sg):
        off = pl.multiple_of(ebase + c * CH, CH)
        pltpu.async_copy(msgs.at[pl.ds(off, CH)], rb, sg)

    def gwait(rb, sg):
        pltpu.make_async_copy(msgs.at[pl.ds(0, CH)], rb, sg).wait()

    g(0, rb0, sg0)
    g(1, rb1, sg1)

    def pair(j, _):
        c0 = j * 2
        c1 = c0 + 1
        gwait(rb0, sg0)
        pltpu.sync_copy(rb0, aggsh.at[didxa.at[c0]], add=True)
        g(jnp.minimum(c0 + 2, last), rb0, sg0)
        gwait(rb1, sg1)
        pltpu.sync_copy(rb1, aggsh.at[didxa.at[c1]], add=True)
        g(jnp.minimum(c1 + 2, last), rb1, sg1)
        return 0

    lax.fori_loop(0, nch // 2, pair, 0)
    gwait(rb0, sg0)
    gwait(rb1, sg1)
    plsc.subcore_barrier()
    rp = nrows // NS
    pltpu.sync_copy(aggsh.at[pl.ds(sid * rp, rp)],
                    out.at[cid, pl.ds(sid * rp, rp)])


def _scatadd(msgs, dstp3, nrows, e_pad, width, nch):
    epw = e_pad // NW
    zinit = jnp.zeros((nrows, width), jnp.float32)
    body = functools.partial(_scatadd_body, nch=nch, epw=epw,
                             nrows=nrows, width=width)
    f = pl.kernel(
        body,
        out_type=jax.ShapeDtypeStruct((NC, nrows, width), jnp.float32),
        mesh=_MESH,
        scratch_types=[
            pltpu.VMEM((nch, CH), jnp.int32),
            pltpu.VMEM((CH, width), jnp.float32),
            pltpu.VMEM((CH, width), jnp.float32),
            pltpu.VMEM_SHARED((nrows, width), jnp.float32),
            pltpu.SemaphoreType.DMA,
            pltpu.SemaphoreType.DMA,
        ],
    )
    return f(msgs, dstp3, zinit)


def _agg_body(table, srcp, dstp3, zinit, out,
              sidx0, sidx1, didxa, rb0, rb1,
              aggsh, si0, si1, sg0, sg1, *, nch, epw, nrows):
    cid = lax.axis_index("c")
    sid = lax.axis_index("s")
    wid = sid * NC + cid
    ebase = wid * epw
    last = nch - 1

    @pl.when(sid == 0)
    def _():
        pltpu.sync_copy(zinit, aggsh)
    pltpu.sync_copy(dstp3.at[wid], didxa)
    plsc.subcore_barrier()

    def idx(c, sx, si):
        off = pl.multiple_of(ebase + c * CH, CH)
        pltpu.async_copy(srcp.at[pl.ds(off, CH)], sx, si)

    def iwait(sx, si):
        pltpu.make_async_copy(srcp.at[pl.ds(0, CH)], sx, si).wait()

    def g(sx, rb, sg):
        pltpu.async_copy(table.at[sx], rb, sg)

    def gwait(rb, sg):
        pltpu.make_async_copy(table.at[pl.ds(0, CH)], rb, sg).wait()

    idx(0, sidx0, si0)
    idx(1, sidx1, si1)
    iwait(sidx0, si0)
    g(sidx0, rb0, sg0)

    def pair(j, _):
        c0 = j * 2
        c1 = c0 + 1
        iwait(sidx1, si1)
        g(sidx1, rb1, sg1)
        gwait(rb0, sg0)
        pltpu.sync_copy(rb0, aggsh.at[didxa.at[c0]], add=True)
        idx(jnp.minimum(c0 + 2, last), sidx0, si0)
        iwait(sidx0, si0)
        g(sidx0, rb0, sg0)
        gwait(rb1, sg1)
        pltpu.sync_copy(rb1, aggsh.at[didxa.at[c1]], add=True)
        idx(jnp.minimum(c1 + 2, last), sidx1, si1)
        return 0

    lax.fori_loop(0, nch // 2, pair, 0)
    gwait(rb0, sg0)
    iwait(sidx1, si1)
    plsc.subcore_barrier()
    rp = nrows // NS
    pltpu.sync_copy(aggsh.at[pl.ds(sid * rp, rp)],
                    out.at[cid, pl.ds(sid * rp, rp)])


def _agg_pass(table, srcp, dstp3, nrows, e_pad, nch):
    epw = e_pad // NW
    zinit = jnp.zeros((nrows, W128), jnp.float32)
    body = functools.partial(_agg_body, nch=nch, epw=epw, nrows=nrows)
    f = pl.kernel(
        body,
        out_type=jax.ShapeDtypeStruct((NC, nrows, W128), jnp.float32),
        mesh=_MESH,
        scratch_types=[
            pltpu.VMEM((CH,), jnp.int32),
            pltpu.VMEM((CH,), jnp.int32),
            pltpu.VMEM((nch, CH), jnp.int32),
            pltpu.VMEM((CH, W128), jnp.float32),
            pltpu.VMEM((CH, W128), jnp.float32),
            pltpu.VMEM_SHARED((nrows, W128), jnp.float32),
            pltpu.SemaphoreType.DMA,
            pltpu.SemaphoreType.DMA,
            pltpu.SemaphoreType.DMA,
            pltpu.SemaphoreType.DMA,
        ],
    )
    return f(table, srcp, dstp3, zinit)


def _msg1_body(pe5, p0, p1, zp, mw, mb, out):
    z = jnp.max(zp[...])
    agg = (p0[...][:NPOS] + p1[...][:NPOS]) / z
    pen = pe5[...] + agg
    m = jnp.maximum(pen @ mw[...] + mb[...], 0.0)
    top = jnp.concatenate(
        [m, jnp.ones((NPOS, 16), jnp.float32),
         jnp.zeros((NPOS, W128 - EMB - 16), jnp.float32)], axis=1)
    out[...] = jnp.concatenate(
        [top, jnp.zeros((POSR - NPOS, W128), jnp.float32)], axis=0)


def _msg1(pe5, p0, p1, zp, mw, mb):
    return pl.pallas_call(
        _msg1_body,
        out_shape=jax.ShapeDtypeStruct((POSR, W128), jnp.float32),
    )(pe5, p0, p1, zp, mw, mb)


def _msg2_body(p0, p1, idx, ptab, gsel, esel, u1, u2, ub, tw, tb, out):
    s5 = (p0[...] + p1[...])[:NPOS]
    cnt = jnp.maximum(s5 @ esel[...], 1.0)
    agg = (s5 @ gsel[...]) / cnt
    c2 = agg @ u2[...] + ub[...]
    oh = (idx[...] == lax.broadcasted_iota(jnp.int32, (NPOS, 16), 1)
          ).astype(jnp.float32)
    pos = oh @ ptab[...]
    for _ in range(3):
        pos = jnp.maximum(pos @ u1[...] + c2, 0.0)
    m = jnp.maximum(pos[:NTEAM] @ tw[...] + tb[...], 0.0)
    top = jnp.concatenate(
        [m, jnp.ones((NTEAM, 16), jnp.float32),
         jnp.zeros((NTEAM, W128 - EMB - 16), jnp.float32)], axis=1)
    out[...] = jnp.concatenate(
        [top, jnp.zeros((TEAMR - NTEAM, W128), jnp.float32)], axis=0)


def _msg2(p0, p1, idx, ptab, gsel, esel, u1, u2, ub, tw, tb):
    return pl.pallas_call(
        _msg2_body,
        out_shape=jax.ShapeDtypeStruct((TEAMR, W128), jnp.float32),
    )(p0, p1, idx, ptab, gsel, esel, u1, u2, ub, tw, tb)


def _team_body(p0, p1, idx, ttab, gsel, esel, u1, u2, ub, out):
    s = (p0[...] + p1[...])[:NTEAM]
    cnt = jnp.maximum(s @ esel[...], 1.0)
    agg = (s @ gsel[...]) / cnt
    c2 = agg @ u2[...] + ub[...]
    oh = (idx[...] == lax.broadcasted_iota(jnp.int32, (NTEAM, 32), 1)
          ).astype(jnp.float32)
    team = oh @ ttab[...]
    for _ in range(3):
        team = jnp.maximum(team @ u1[...] + c2, 0.0)
    out[...] = team


def _team(p0, p1, idx, ttab, gsel, esel, u1, u2, ub):
    return pl.pallas_call(
        _team_body,
        out_shape=jax.ShapeDtypeStruct((NTEAM, EMB), jnp.float32),
    )(p0, p1, idx, ttab, gsel, esel, u1, u2, ub)


def _pad_edges(arr, n_pad, fill):
    return jnp.concatenate(
        [arr.astype(jnp.int32),
         jnp.full((n_pad - arr.shape[0],), fill, jnp.int32)])


@jax.jit
def kernel(player_features, position_indices, team_indices,
           player_to_position_edges, position_to_team_edges, chemistry_edges,
           enc_W1, enc_b1, enc_W2, enc_b2,
           attn_Wq, attn_bq, attn_Wk, attn_bk, attn_Wv, attn_bv,
           pos_table, team_table,
           p2p_msg_W, p2p_msg_b, p2p_upd_W, p2p_upd_b,
           p2t_msg_W, p2t_msg_b, p2t_upd_W, p2t_upd_b):
    r2 = lambda b: b.reshape(1, -1)

    pe, t1, t2 = _encode_qkv(
        player_features, enc_W1, r2(enc_b1), enc_W2, r2(enc_b2),
        attn_Wq, r2(attn_bq), attn_Wk, r2(attn_bk), attn_Wv, r2(attn_bv))

    csrc = _pad_edges(chemistry_edges[0], E_CHEM_PAD, 0)
    cdst = _pad_edges(chemistry_edges[1], E_CHEM_PAD, 0)
    cdst_cl = jnp.minimum(cdst, NPOS).reshape(NW, NCH_CHEM, CH)
    qg, kvg = _gather2(t1, t2, csrc, cdst)
    msgs, zp = _wmsg(qg, kvg)
    aggc = _scatadd(msgs, cdst_cl, POSR, E_CHEM_PAD, EMB, NCH_CHEM)

    mp = _msg1(pe[:NPOS], aggc[0], aggc[1], zp,
               p2p_msg_W, r2(p2p_msg_b))

    psrc = _pad_edges(player_to_position_edges[0], E_P2P_PAD, 0)
    pdst = _pad_edges(player_to_position_edges[1], E_P2P_PAD, NPOS
                      ).reshape(NW, NCH_CHEM, CH)
    aggp = _agg_pass(mp, psrc, pdst, POSR, E_P2P_PAD, NCH_CHEM)

    gsel = jnp.concatenate(
        [jnp.eye(EMB, dtype=jnp.float32),
         jnp.zeros((W128 - EMB, EMB), jnp.float32)], axis=0)
    esel = jnp.concatenate(
        [jnp.zeros((EMB, EMB), jnp.float32),
         jnp.full((16, EMB), 1.0 / 16.0, jnp.float32),
         jnp.zeros((W128 - EMB - 16, EMB), jnp.float32)], axis=0)

    ptab16 = jnp.concatenate(
        [pos_table, jnp.zeros((6, EMB), jnp.float32)], axis=0)
    mt = _msg2(aggp[0], aggp[1], position_indices.astype(jnp.int32)[:, None],
               ptab16, gsel, esel,
               p2p_upd_W[:EMB], p2p_upd_W[EMB:], r2(p2p_upd_b),
               p2t_msg_W, r2(p2t_msg_b))

    tsrc = _pad_edges(position_to_team_edges[0], E_P2T_PAD, 0)
    tdst = _pad_edges(position_to_team_edges[1], E_P2T_PAD, NTEAM
                      ).reshape(NW, NCH_P2T, CH)
    aggt = _agg_pass(mt, tsrc, tdst, TEAMR, E_P2T_PAD, NCH_P2T)

    return _team(aggt[0], aggt[1], team_indices.astype(jnp.int32)[:, None],
                 team_table, gsel, esel,
                 p2t_upd_W[:EMB], p2t_upd_W[EMB:], r2(p2t_upd_b))


# speedup vs baseline: 1.1744x; 1.0850x over previous
import functools

import jax
import jax.numpy as jnp
from jax import lax
from jax.experimental import pallas as pl
from jax.experimental.pallas import tpu as pltpu
from jax.experimental.pallas import tpu_sc as plsc

NPL = 50000
NPOS = 5000
NTEAM = 320
FEAT = 128
HID = 128
EMB = 64
E_CHEM = 800000
E_P2P = 800000
E_P2T = 80000

NC, NS, L = 2, 16, 16
NW = NC * NS
CH = 128
W128 = 128

POSR = 5120
TEAMR = 384

NCH_CHEM = 196
NCH_P2T = 20
E_CHEM_PAD = NW * CH * NCH_CHEM
E_P2P_PAD = NW * CH * NCH_CHEM
E_P2T_PAD = NW * CH * NCH_P2T

_MESH = plsc.VectorSubcoreMesh(core_axis_name="c", subcore_axis_name="s")


def _enc_body(pf, w1, b1, w2, b2, wq, bq, wk, bk, wv, bv,
              pe_o, t1_o, t2_o):
    h = jnp.maximum(pf[...] @ w1[...] + b1[...], 0.0)
    pe = h @ w2[...] + b2[...]
    pe_o[...] = pe
    q = pe @ wq[...] + bq[...]
    k = pe @ wk[...] + bk[...]
    v = pe @ wv[...] + bv[...]
    t1_o[...] = jnp.concatenate(
        [q, jnp.zeros(q.shape, jnp.float32)], axis=1)
    t2_o[...] = jnp.concatenate([k, v], axis=1)


def _encode_qkv(pf, w1, b1, w2, b2, wq, bq, wk, bk, wv, bv):
    blk = 2000
    grid = NPL // blk
    full = lambda a: pl.BlockSpec(a.shape, lambda i: (0,) * a.ndim)
    return pl.pallas_call(
        _enc_body,
        grid=(grid,),
        in_specs=[pl.BlockSpec((blk, FEAT), lambda i: (i, 0)),
                  full(w1), full(b1), full(w2), full(b2),
                  full(wq), full(bq), full(wk), full(bk), full(wv), full(bv)],
        out_specs=[pl.BlockSpec((blk, EMB), lambda i: (i, 0)),
                   pl.BlockSpec((blk, W128), lambda i: (i, 0)),
                   pl.BlockSpec((blk, W128), lambda i: (i, 0))],
        out_shape=[jax.ShapeDtypeStruct((NPL, EMB), jnp.float32),
                   jax.ShapeDtypeStruct((NPL, W128), jnp.float32),
                   jax.ShapeDtypeStruct((NPL, W128), jnp.float32)],
    )(pf, w1, b1, w2, b2, wq, bq, wk, bk, wv, bv)


_CHEM_EPW = E_CHEM_PAD // NW


def _gather2_body(t1, t2, srcp, dstp,
                  qg, kvg,
                  sidx0, sidx1, didx0, didx1, qb0, qb1, kvb0, kvb1,
                  si0, si1, sg0, sg1, sw0, sw1):
    cid = lax.axis_index("c")
    sid = lax.axis_index("s")
    wid = sid * NC + cid
    ebase = wid * _CHEM_EPW
    last = NCH_CHEM - 1

    def idx(c, sx, dx, si):
        off = pl.multiple_of(ebase + c * CH, CH)
        pltpu.async_copy(srcp.at[pl.ds(off, CH)], sx, si)
        pltpu.async_copy(dstp.at[pl.ds(off, CH)], dx, si)

    def iwait(sx, dx, si):
        pltpu.make_async_copy(srcp.at[pl.ds(0, CH)], sx, si).wait()
        pltpu.make_async_copy(dstp.at[pl.ds(0, CH)], dx, si).wait()

    def g(sx, dx, qb, kvb, sg):
        pltpu.async_copy(t1.at[dx], qb, sg)
        pltpu.async_copy(t2.at[sx], kvb, sg)

    def gwait(qb, kvb, sg):
        pltpu.make_async_copy(t1.at[pl.ds(0, CH)], qb, sg).wait()
        pltpu.make_async_copy(t2.at[pl.ds(0, CH)], kvb, sg).wait()

    def w(c, qb, kvb, sw):
        off = pl.multiple_of(ebase + c * CH, CH)
        pltpu.async_copy(qb, qg.at[pl.ds(off, CH)], sw)
        pltpu.async_copy(kvb, kvg.at[pl.ds(off, CH)], sw)

    def wwait(qb, kvb, sw):
        pltpu.make_async_copy(qb, qg.at[pl.ds(0, CH)], sw).wait()
        pltpu.make_async_copy(kvb, kvg.at[pl.ds(0, CH)], sw).wait()

    idx(0, sidx0, didx0, si0)
    idx(1, sidx1, didx1, si1)
    iwait(sidx0, didx0, si0)
    g(sidx0, didx0, qb0, kvb0, sg0)

    def pair(j, _):
        c0 = j * 2
        c1 = c0 + 1
        iwait(sidx1, didx1, si1)
        g(sidx1, didx1, qb1, kvb1, sg1)
        gwait(qb0, kvb0, sg0)
        w(c0, qb0, kvb0, sw0)
        wwait(qb0, kvb0, sw0)
        idx(jnp.minimum(c0 + 2, last), sidx0, didx0, si0)
        iwait(sidx0, didx0, si0)
        g(sidx0, didx0, qb0, kvb0, sg0)
        gwait(qb1, kvb1, sg1)
        w(c1, qb1, kvb1, sw1)
        wwait(qb1, kvb1, sw1)
        idx(jnp.minimum(c1 + 2, last), sidx1, didx1, si1)
        return 0

    lax.fori_loop(0, NCH_CHEM // 2, pair, 0)
    gwait(qb0, kvb0, sg0)
    iwait(sidx1, didx1, si1)


def _gather2(t1, t2, srcp, dstp):
    eg = jax.ShapeDtypeStruct((E_CHEM_PAD, W128), jnp.float32)
    f = pl.kernel(
        _gather2_body,
        out_type=(eg, eg),
        mesh=_MESH,
        scratch_types=[
            pltpu.VMEM((CH,), jnp.int32),
            pltpu.VMEM((CH,), jnp.int32),
            pltpu.VMEM((CH,), jnp.int32),
            pltpu.VMEM((CH,), jnp.int32),
            pltpu.VMEM((CH, W128), jnp.float32),
            pltpu.VMEM((CH, W128), jnp.float32),
            pltpu.VMEM((CH, W128), jnp.float32),
            pltpu.VMEM((CH, W128), jnp.float32),
            pltpu.SemaphoreType.DMA,
            pltpu.SemaphoreType.DMA,
            pltpu.SemaphoreType.DMA,
            pltpu.SemaphoreType.DMA,
            pltpu.SemaphoreType.DMA,
            pltpu.SemaphoreType.DMA,
        ],
    )
    return f(t1, t2, srcp, dstp)


_WB = 8192
_WGRID = E_CHEM_PAD // _WB


def _wmsg_body(qg, kvg, ones64, m_o, z_o):
    i = pl.program_id(0)
    inv = 1.0 / (EMB ** 0.5)
    q = qg[...][:, :EMB]
    kv = kvg[...]
    k = kv[:, :EMB]
    v = kv[:, EMB:]
    s = ((q * k) @ ones64[...]) * inv
    row = lax.broadcasted_iota(jnp.int32, (_WB, EMB), 0) + i * _WB
    w = jnp.where(row < E_CHEM, jnp.exp(s), 0.0)
    m_o[...] = w * v
    zb = jnp.sum(w) * (1.0 / EMB)

    @pl.when(i == 0)
    def _():
        z_o[...] = jnp.zeros((8, 128), jnp.float32)

    z_o[...] += jnp.full((8, 128), zb, jnp.float32)


def _wmsg(qg, kvg):
    ones64 = jnp.ones((EMB, EMB), jnp.float32)
    return pl.pallas_call(
        _wmsg_body,
        grid=(_WGRID,),
        in_specs=[pl.BlockSpec((_WB, W128), lambda i: (i, 0)),
                  pl.BlockSpec((_WB, W128), lambda i: (i, 0)),
                  pl.BlockSpec((EMB, EMB), lambda i: (0, 0))],
        out_specs=[pl.BlockSpec((_WB, EMB), lambda i: (i, 0)),
                   pl.BlockSpec((8, 128), lambda i: (0, 0))],
        out_shape=[jax.ShapeDtypeStruct((E_CHEM_PAD, EMB), jnp.float32),
                   jax.ShapeDtypeStruct((8, 128), jnp.float32)],
    )(qg, kvg, ones64)


def _scatadd_body(msgs, dstp3, zinit, out, didxa, rb0, rb1, aggsh,
                  sg0, sg1, *, nch, epw, nrows, width):
    cid = lax.axis_index("c")
    sid = lax.axis_index("s")
    wid = sid * NC + cid
    ebase = wid * epw

    @pl.when(sid == 0)
    def _():
        pltpu.sync_copy(zinit, aggsh)
    pltpu.sync_copy(dstp3.at[wid], didxa)
    plsc.subcore_barrier()
    last = nch - 1

    def g(c, rb, sg):
        off = pl.multiple_of(ebase + c * CH, CH)
        pltpu.async_copy(msgs.at[pl.ds(off, CH)], rb, sg)

    def gwait(rb, sg):
        pltpu.make_async_copy(msgs.at[pl.ds(0, CH)], rb, sg).wait()

    g(0, rb0, sg0)
    g(1, rb1, sg1)

    def pair(j, _):
        c0 = j * 2
        c1 = c0 + 1
        gwait(rb0, sg0)
        pltpu.sync_copy(rb0, aggsh.at[didxa.at[c0]], add=True)
        g(jnp.minimum(c0 + 2, last), rb0, sg0)
        gwait(rb1, sg1)
        pltpu.sync_copy(rb1, aggsh.at[didxa.at[c1]], add=True)
        g(jnp.minimum(c1 + 2, last), rb1, sg1)
        return 0

    lax.fori_loop(0, nch // 2, pair, 0)
    gwait(rb0, sg0)
    gwait(rb1, sg1)
    plsc.subcore_barrier()
    rp = nrows // NS
    pltpu.sync_copy(aggsh.at[pl.ds(sid * rp, rp)],
                    out.at[cid, pl.ds(sid * rp, rp)])


def _scatadd(msgs, dstp3, nrows, e_pad, width, nch):
    epw = e_pad // NW
    zinit = jnp.zeros((nrows, width), jnp.float32)
    body = functools.partial(_scatadd_body, nch=nch, epw=epw,
                             nrows=nrows, width=width)
    f = pl.kernel(
        body,
        out_type=jax.ShapeDtypeStruct((NC, nrows, width), jnp.float32),
        mesh=_MESH,
        scratch_types=[
            pltpu.VMEM((nch, CH), jnp.int32),
            pltpu.VMEM((CH, width), jnp.float32),
            pltpu.VMEM((CH, width), jnp.float32),
            pltpu.VMEM_SHARED((nrows, width), jnp.float32),
            pltpu.SemaphoreType.DMA,
            pltpu.SemaphoreType.DMA,
        ],
    )
    return f(msgs, dstp3, zinit)


def _agg_body(table, srcp, dstp3, zinit, out,
              sidx0, sidx1, didxa, rb0, rb1,
              aggsh, si0, si1, sg0, sg1, *, nch, epw, nrows):
    cid = lax.axis_index("c")
    sid = lax.axis_index("s")
    wid = sid * NC + cid
    ebase = wid * epw
    last = nch - 1

    @pl.when(sid == 0)
    def _():
        pltpu.sync_copy(zinit, aggsh)
    pltpu.sync_copy(dstp3.at[wid], didxa)
    plsc.subcore_barrier()

    def idx(c, sx, si):
        off = pl.multiple_of(ebase + c * CH, CH)
        pltpu.async_copy(srcp.at[pl.ds(off, CH)], sx, si)

    def iwait(sx, si):
        pltpu.make_async_copy(srcp.at[pl.ds(0, CH)], sx, si).wait()

    def g(sx, rb, sg):
        pltpu.async_copy(table.at[sx], rb, sg)

    def gwait(rb, sg):
        pltpu.make_async_copy(table.at[pl.ds(0, CH)], rb, sg).wait()

    idx(0, sidx0, si0)
    idx(1, sidx1, si1)
    iwait(sidx0, si0)
    g(sidx0, rb0, sg0)

    def pair(j, _):
        c0 = j * 2
        c1 = c0 + 1
        iwait(sidx1, si1)
        g(sidx1, rb1, sg1)
        gwait(rb0, sg0)
        pltpu.sync_copy(rb0, aggsh.at[didxa.at[c0]], add=True)
        idx(jnp.minimum(c0 + 2, last), sidx0, si0)
        iwait(sidx0, si0)
        g(sidx0, rb0, sg0)
        gwait(rb1, sg1)
        pltpu.sync_copy(rb1, aggsh.at[didxa.at[c1]], add=True)
        idx(jnp.minimum(c1 + 2, last), sidx1, si1)
        return 0

    lax.fori_loop(0, nch // 2, pair, 0)
    gwait(rb0, sg0)
    iwait(sidx1, si1)
    plsc.subcore_barrier()
    rp = nrows // NS
    pltpu.sync_copy(aggsh.at[pl.ds(sid * rp, rp)],
                    out.at[cid, pl.ds(sid * rp, rp)])


def _agg_pass(table, srcp, dstp3, nrows, e_pad, nch):
    epw = e_pad // NW
    zinit = jnp.zeros((nrows, W128), jnp.float32)
    body = functools.partial(_agg_body, nch=nch, epw=epw, nrows=nrows)
    f = pl.kernel(
        body,
        out_type=jax.ShapeDtypeStruct((NC, nrows, W128), jnp.float32),
        mesh=_MESH,
        scratch_types=[
            pltpu.VMEM((CH,), jnp.int32),
            pltpu.VMEM((CH,), jnp.int32),
            pltpu.VMEM((nch, CH), jnp.int32),
            pltpu.VMEM((CH, W128), jnp.float32),
            pltpu.VMEM((CH, W128), jnp.float32),
            pltpu.VMEM_SHARED((nrows, W128), jnp.float32),
            pltpu.SemaphoreType.DMA,
            pltpu.SemaphoreType.DMA,
            pltpu.SemaphoreType.DMA,
            pltpu.SemaphoreType.DMA,
        ],
    )
    return f(table, srcp, dstp3, zinit)


def _msg1_body(pe5, p0, p1, zp, mw, mb, out):
    z = jnp.max(zp[...])
    agg = (p0[...][:NPOS] + p1[...][:NPOS]) / z
    pen = pe5[...] + agg
    m = jnp.maximum(pen @ mw[...] + mb[...], 0.0)
    top = jnp.concatenate(
        [m, jnp.ones((NPOS, 16), jnp.float32),
         jnp.zeros((NPOS, W128 - EMB - 16), jnp.float32)], axis=1)
    out[...] = jnp.concatenate(
        [top, jnp.zeros((POSR - NPOS, W128), jnp.float32)], axis=0)


def _msg1(pe5, p0, p1, zp, mw, mb):
    return pl.pallas_call(
        _msg1_body,
        out_shape=jax.ShapeDtypeStruct((POSR, W128), jnp.float32),
    )(pe5, p0, p1, zp, mw, mb)


def _msg2_body(p0, p1, idx, ptab, gsel, esel, u1, u2, ub, tw, tb, out):
    s5 = (p0[...] + p1[...])[:NPOS]
    cnt = jnp.maximum(s5 @ esel[...], 1.0)
    agg = (s5 @ gsel[...]) / cnt
    c2 = agg @ u2[...] + ub[...]
    oh = (idx[...] == lax.broadcasted_iota(jnp.int32, (NPOS, 16), 1)
          ).astype(jnp.float32)
    pos = oh @ ptab[...]
    for _ in range(3):
        pos = jnp.maximum(pos @ u1[...] + c2, 0.0)
    m = jnp.maximum(pos[:NTEAM] @ tw[...] + tb[...], 0.0)
    top = jnp.concatenate(
        [m, jnp.ones((NTEAM, 16), jnp.float32),
         jnp.zeros((NTEAM, W128 - EMB - 16), jnp.float32)], axis=1)
    out[...] = jnp.concatenate(
        [top, jnp.zeros((TEAMR - NTEAM, W128), jnp.float32)], axis=0)


def _msg2(p0, p1, idx, ptab, gsel, esel, u1, u2, ub, tw, tb):
    return pl.pallas_call(
        _msg2_body,
        out_shape=jax.ShapeDtypeStruct((TEAMR, W128), jnp.float32),
    )(p0, p1, idx, ptab, gsel, esel, u1, u2, ub, tw, tb)


def _team_body(p0, p1, idx, ttab, gsel, esel, u1, u2, ub, out):
    s = (p0[...] + p1[...])[:NTEAM]
    cnt = jnp.maximum(s @ esel[...], 1.0)
    agg = (s @ gsel[...]) / cnt
    c2 = agg @ u2[...] + ub[...]
    oh = (idx[...] == lax.broadcasted_iota(jnp.int32, (NTEAM, 32), 1)
          ).astype(jnp.float32)
    team = oh @ ttab[...]
    for _ in range(3):
        team = jnp.maximum(team @ u1[...] + c2, 0.0)
    out[...] = team


def _team(p0, p1, idx, ttab, gsel, esel, u1, u2, ub):
    return pl.pallas_call(
        _team_body,
        out_shape=jax.ShapeDtypeStruct((NTEAM, EMB), jnp.float32),
    )(p0, p1, idx, ttab, gsel, esel, u1, u2, ub)


def _pad_edges(arr, n_pad, fill):
    return jnp.concatenate(
        [arr.astype(jnp.int32),
         jnp.full((n_pad - arr.shape[0],), fill, jnp.int32)])


@jax.jit
def kernel(player_features, position_indices, team_indices,
           player_to_position_edges, position_to_team_edges, chemistry_edges,
           enc_W1, enc_b1, enc_W2, enc_b2,
           attn_Wq, attn_bq, attn_Wk, attn_bk, attn_Wv, attn_bv,
           pos_table, team_table,
           p2p_msg_W, p2p_msg_b, p2p_upd_W, p2p_upd_b,
           p2t_msg_W, p2t_msg_b, p2t_upd_W, p2t_upd_b):
    r2 = lambda b: b.reshape(1, -1)

    pe, t1, t2 = _encode_qkv(
        player_features, enc_W1, r2(enc_b1), enc_W2, r2(enc_b2),
        attn_Wq, r2(attn_bq), attn_Wk, r2(attn_bk), attn_Wv, r2(attn_bv))

    csrc = _pad_edges(chemistry_edges[0], E_CHEM_PAD, 0)
    cdst = _pad_edges(chemistry_edges[1], E_CHEM_PAD, 0)
    cdst_cl = jnp.minimum(cdst, NPOS).reshape(NW, NCH_CHEM, CH)
    qg, kvg = _gather2(t1, t2, csrc, cdst)
    msgs, zp = _wmsg(qg, kvg)
    aggc = _scatadd(msgs, cdst_cl, POSR, E_CHEM_PAD, EMB, NCH_CHEM)

    mp = _msg1(pe[:NPOS], aggc[0], aggc[1], zp,
               p2p_msg_W, r2(p2p_msg_b))

    psrc = _pad_edges(player_to_position_edges[0], E_P2P_PAD, 0)
    pdst = _pad_edges(player_to_position_edges[1], E_P2P_PAD, NPOS
                      ).reshape(NW, NCH_CHEM, CH)
    aggp = _agg_pass(mp, psrc, pdst, POSR, E_P2P_PAD, NCH_CHEM)

    gsel = jnp.concatenate(
        [jnp.eye(EMB, dtype=jnp.float32),
         jnp.zeros((W128 - EMB, EMB), jnp.float32)], axis=0)
    esel = jnp.concatenate(
        [jnp.zeros((EMB, EMB), jnp.float32),
         jnp.full((16, EMB), 1.0 / 16.0, jnp.float32),
         jnp.zeros((W128 - EMB - 16, EMB), jnp.float32)], axis=0)

    ptab16 = jnp.concatenate(
        [pos_table, jnp.zeros((6, EMB), jnp.float32)], axis=0)
    mt = _msg2(aggp[0], aggp[1], position_indices.astype(jnp.int32)[:, None],
               ptab16, gsel, esel,
               p2p_upd_W[:EMB], p2p_upd_W[EMB:], r2(p2p_upd_b),
               p2t_msg_W, r2(p2t_msg_b))

    tsrc = _pad_edges(position_to_team_edges[0], E_P2T_PAD, 0)
    tdst = _pad_edges(position_to_team_edges[1], E_P2T_PAD, NTEAM
                      ).reshape(NW, NCH_P2T, CH)
    aggt = _agg_pass(mt, tsrc, tdst, TEAMR, E_P2T_PAD, NCH_P2T)

    return _team(aggt[0], aggt[1], team_indices.astype(jnp.int32)[:, None],
                 team_table, gsel, esel,
                 p2t_upd_W[:EMB], p2t_upd_W[EMB:], r2(p2t_upd_b))


# speedup vs baseline: 1.1808x; 1.0055x over previous
import functools

import jax
import jax.numpy as jnp
from jax import lax
from jax.experimental import pallas as pl
from jax.experimental.pallas import tpu as pltpu
from jax.experimental.pallas import tpu_sc as plsc

NPL = 50000
NPOS = 5000
NTEAM = 320
FEAT = 128
HID = 128
EMB = 64
E_CHEM = 800000
E_P2P = 800000
E_P2T = 80000

NC, NS, L = 2, 16, 16
NW = NC * NS
CH = 128
W128 = 128

POSR = 5120
TEAMR = 384

NCH_CHEM = 196
NCH_P2T = 20
E_CHEM_PAD = NW * CH * NCH_CHEM
E_P2P_PAD = NW * CH * NCH_CHEM
E_P2T_PAD = NW * CH * NCH_P2T

_MESH = plsc.VectorSubcoreMesh(core_axis_name="c", subcore_axis_name="s")


def _enc_body(pf, w1, b1, w2, b2, wq, bq, wk, bk, wv, bv,
              pe_o, t1_o, t2_o):
    h = jnp.maximum(pf[...] @ w1[...] + b1[...], 0.0)
    pe = h @ w2[...] + b2[...]
    pe_o[...] = pe
    q = pe @ wq[...] + bq[...]
    k = pe @ wk[...] + bk[...]
    v = pe @ wv[...] + bv[...]
    t1_o[...] = jnp.concatenate(
        [q, jnp.zeros(q.shape, jnp.float32)], axis=1)
    t2_o[...] = jnp.concatenate([k, v], axis=1)


def _encode_qkv(pf, w1, b1, w2, b2, wq, bq, wk, bk, wv, bv):
    blk = 5000
    grid = NPL // blk
    full = lambda a: pl.BlockSpec(a.shape, lambda i: (0,) * a.ndim)
    return pl.pallas_call(
        _enc_body,
        grid=(grid,),
        in_specs=[pl.BlockSpec((blk, FEAT), lambda i: (i, 0)),
                  full(w1), full(b1), full(w2), full(b2),
                  full(wq), full(bq), full(wk), full(bk), full(wv), full(bv)],
        out_specs=[pl.BlockSpec((blk, EMB), lambda i: (i, 0)),
                   pl.BlockSpec((blk, W128), lambda i: (i, 0)),
                   pl.BlockSpec((blk, W128), lambda i: (i, 0))],
        out_shape=[jax.ShapeDtypeStruct((NPL, EMB), jnp.float32),
                   jax.ShapeDtypeStruct((NPL, W128), jnp.float32),
                   jax.ShapeDtypeStruct((NPL, W128), jnp.float32)],
    )(pf, w1, b1, w2, b2, wq, bq, wk, bk, wv, bv)


_CHEM_EPW = E_CHEM_PAD // NW


def _gather2_body(t1, t2, srcp, dstp,
                  qg, kvg,
                  sidx0, sidx1, didx0, didx1, qb0, qb1, kvb0, kvb1,
                  si0, si1, sg0, sg1, sw0, sw1):
    cid = lax.axis_index("c")
    sid = lax.axis_index("s")
    wid = sid * NC + cid
    ebase = wid * _CHEM_EPW
    last = NCH_CHEM - 1

    def idx(c, sx, dx, si):
        off = pl.multiple_of(ebase + c * CH, CH)
        pltpu.async_copy(srcp.at[pl.ds(off, CH)], sx, si)
        pltpu.async_copy(dstp.at[pl.ds(off, CH)], dx, si)

    def iwait(sx, dx, si):
        pltpu.make_async_copy(srcp.at[pl.ds(0, CH)], sx, si).wait()
        pltpu.make_async_copy(dstp.at[pl.ds(0, CH)], dx, si).wait()

    def g(sx, dx, qb, kvb, sg):
        pltpu.async_copy(t1.at[dx], qb, sg)
        pltpu.async_copy(t2.at[sx], kvb, sg)

    def gwait(qb, kvb, sg):
        pltpu.make_async_copy(t1.at[pl.ds(0, CH)], qb, sg).wait()
        pltpu.make_async_copy(t2.at[pl.ds(0, CH)], kvb, sg).wait()

    def w(c, qb, kvb, sw):
        off = pl.multiple_of(ebase + c * CH, CH)
        pltpu.async_copy(qb, qg.at[pl.ds(off, CH)], sw)
        pltpu.async_copy(kvb, kvg.at[pl.ds(off, CH)], sw)

    def wwait(qb, kvb, sw):
        pltpu.make_async_copy(qb, qg.at[pl.ds(0, CH)], sw).wait()
        pltpu.make_async_copy(kvb, kvg.at[pl.ds(0, CH)], sw).wait()

    idx(0, sidx0, didx0, si0)
    idx(1, sidx1, didx1, si1)
    iwait(sidx0, didx0, si0)
    g(sidx0, didx0, qb0, kvb0, sg0)

    def pair(j, _):
        c0 = j * 2
        c1 = c0 + 1
        iwait(sidx1, didx1, si1)
        g(sidx1, didx1, qb1, kvb1, sg1)
        gwait(qb0, kvb0, sg0)
        w(c0, qb0, kvb0, sw0)
        wwait(qb0, kvb0, sw0)
        idx(jnp.minimum(c0 + 2, last), sidx0, didx0, si0)
        iwait(sidx0, didx0, si0)
        g(sidx0, didx0, qb0, kvb0, sg0)
        gwait(qb1, kvb1, sg1)
        w(c1, qb1, kvb1, sw1)
        wwait(qb1, kvb1, sw1)
        idx(jnp.minimum(c1 + 2, last), sidx1, didx1, si1)
        return 0

    lax.fori_loop(0, NCH_CHEM // 2, pair, 0)
    gwait(qb0, kvb0, sg0)
    iwait(sidx1, didx1, si1)


def _gather2(t1, t2, srcp, dstp):
    eg = jax.ShapeDtypeStruct((E_CHEM_PAD, W128), jnp.float32)
    f = pl.kernel(
        _gather2_body,
        out_type=(eg, eg),
        mesh=_MESH,
        scratch_types=[
            pltpu.VMEM((CH,), jnp.int32),
            pltpu.VMEM((CH,), jnp.int32),
            pltpu.VMEM((CH,), jnp.int32),
            pltpu.VMEM((CH,), jnp.int32),
            pltpu.VMEM((CH, W128), jnp.float32),
            pltpu.VMEM((CH, W128), jnp.float32),
            pltpu.VMEM((CH, W128), jnp.float32),
            pltpu.VMEM((CH, W128), jnp.float32),
            pltpu.SemaphoreType.DMA,
            pltpu.SemaphoreType.DMA,
            pltpu.SemaphoreType.DMA,
            pltpu.SemaphoreType.DMA,
            pltpu.SemaphoreType.DMA,
            pltpu.SemaphoreType.DMA,
        ],
    )
    return f(t1, t2, srcp, dstp)


_WB = 16384
_WGRID = E_CHEM_PAD // _WB


def _wmsg_body(qg, kvg, ones64, m_o, z_o):
    i = pl.program_id(0)
    inv = 1.0 / (EMB ** 0.5)
    q = qg[...][:, :EMB]
    kv = kvg[...]
    k = kv[:, :EMB]
    v = kv[:, EMB:]
    s = ((q * k) @ ones64[...]) * inv
    row = lax.broadcasted_iota(jnp.int32, (_WB, EMB), 0) + i * _WB
    w = jnp.where(row < E_CHEM, jnp.exp(s), 0.0)
    m_o[...] = w * v
    zb = jnp.sum(w) * (1.0 / EMB)

    @pl.when(i == 0)
    def _():
        z_o[...] = jnp.zeros((8, 128), jnp.float32)

    z_o[...] += jnp.full((8, 128), zb, jnp.float32)


def _wmsg(qg, kvg):
    ones64 = jnp.ones((EMB, EMB), jnp.float32)
    return pl.pallas_call(
        _wmsg_body,
        grid=(_WGRID,),
        in_specs=[pl.BlockSpec((_WB, W128), lambda i: (i, 0)),
                  pl.BlockSpec((_WB, W128), lambda i: (i, 0)),
                  pl.BlockSpec((EMB, EMB), lambda i: (0, 0))],
        out_specs=[pl.BlockSpec((_WB, EMB), lambda i: (i, 0)),
                   pl.BlockSpec((8, 128), lambda i: (0, 0))],
        out_shape=[jax.ShapeDtypeStruct((E_CHEM_PAD, EMB), jnp.float32),
                   jax.ShapeDtypeStruct((8, 128), jnp.float32)],
    )(qg, kvg, ones64)


def _scatadd_body(msgs, dstp3, zinit, out, didxa, rb0, rb1, aggsh,
                  sg0, sg1, *, nch, epw, nrows, width):
    cid = lax.axis_index("c")
    sid = lax.axis_index("s")
    wid = sid * NC + cid
    ebase = wid * epw

    @pl.when(sid == 0)
    def _():
        pltpu.sync_copy(zinit, aggsh)
    pltpu.sync_copy(dstp3.at[wid], didxa)
    plsc.subcore_barrier()
    last = nch - 1

    def g(c, rb, sg):
        off = pl.multiple_of(ebase + c * CH, CH)
        pltpu.async_copy(msgs.at[pl.ds(off, CH)], rb, sg)

    def gwait(rb, sg):
        pltpu.make_async_copy(msgs.at[pl.ds(0, CH)], rb, sg).wait()

    g(0, rb0, sg0)
    g(1, rb1, sg1)

    def pair(j, _):
        c0 = j * 2
        c1 = c0 + 1
        gwait(rb0, sg0)
        pltpu.sync_copy(rb0, aggsh.at[didxa.at[c0]], add=True)
        g(jnp.minimum(c0 + 2, last), rb0, sg0)
        gwait(rb1, sg1)
        pltpu.sync_copy(rb1, aggsh.at[didxa.at[c1]], add=True)
        g(jnp.minimum(c1 + 2, last), rb1, sg1)
        return 0

    lax.fori_loop(0, nch // 2, pair, 0)
    gwait(rb0, sg0)
    gwait(rb1, sg1)
    plsc.subcore_barrier()
    rp = nrows // NS
    pltpu.sync_copy(aggsh.at[pl.ds(sid * rp, rp)],
                    out.at[cid, pl.ds(sid * rp, rp)])


def _scatadd(msgs, dstp3, nrows, e_pad, width, nch):
    epw = e_pad // NW
    zinit = jnp.zeros((nrows, width), jnp.float32)
    body = functools.partial(_scatadd_body, nch=nch, epw=epw,
                             nrows=nrows, width=width)
    f = pl.kernel(
        body,
        out_type=jax.ShapeDtypeStruct((NC, nrows, width), jnp.float32),
        mesh=_MESH,
        scratch_types=[
            pltpu.VMEM((nch, CH), jnp.int32),
            pltpu.VMEM((CH, width), jnp.float32),
            pltpu.VMEM((CH, width), jnp.float32),
            pltpu.VMEM_SHARED((nrows, width), jnp.float32),
            pltpu.SemaphoreType.DMA,
            pltpu.SemaphoreType.DMA,
        ],
    )
    return f(msgs, dstp3, zinit)


def _agg_body(table, srcp, dstp3, zinit, out,
              sidx0, sidx1, didxa, rb0, rb1,
              aggsh, si0, si1, sg0, sg1, *, nch, epw, nrows):
    cid = lax.axis_index("c")
    sid = lax.axis_index("s")
    wid = sid * NC + cid
    ebase = wid * epw
    last = nch - 1

    @pl.when(sid == 0)
    def _():
        pltpu.sync_copy(zinit, aggsh)
    pltpu.sync_copy(dstp3.at[wid], didxa)
    plsc.subcore_barrier()

    def idx(c, sx, si):
        off = pl.multiple_of(ebase + c * CH, CH)
        pltpu.async_copy(srcp.at[pl.ds(off, CH)], sx, si)

    def iwait(sx, si):
        pltpu.make_async_copy(srcp.at[pl.ds(0, CH)], sx, si).wait()

    def g(sx, rb, sg):
        pltpu.async_copy(table.at[sx], rb, sg)

    def gwait(rb, sg):
        pltpu.make_async_copy(table.at[pl.ds(0, CH)], rb, sg).wait()

    idx(0, sidx0, si0)
    idx(1, sidx1, si1)
    iwait(sidx0, si0)
    g(sidx0, rb0, sg0)

    def pair(j, _):
        c0 = j * 2
        c1 = c0 + 1
        iwait(sidx1, si1)
        g(sidx1, rb1, sg1)
        gwait(rb0, sg0)
        pltpu.sync_copy(rb0, aggsh.at[didxa.at[c0]], add=True)
        idx(jnp.minimum(c0 + 2, last), sidx0, si0)
        iwait(sidx0, si0)
        g(sidx0, rb0, sg0)
        gwait(rb1, sg1)
        pltpu.sync_copy(rb1, aggsh.at[didxa.at[c1]], add=True)
        idx(jnp.minimum(c1 + 2, last), sidx1, si1)
        return 0

    lax.fori_loop(0, nch // 2, pair, 0)
    gwait(rb0, sg0)
    iwait(sidx1, si1)
    plsc.subcore_barrier()
    rp = nrows // NS
    pltpu.sync_copy(aggsh.at[pl.ds(sid * rp, rp)],
                    out.at[cid, pl.ds(sid * rp, rp)])


def _agg_pass(table, srcp, dstp3, nrows, e_pad, nch):
    epw = e_pad // NW
    zinit = jnp.zeros((nrows, W128), jnp.float32)
    body = functools.partial(_agg_body, nch=nch, epw=epw, nrows=nrows)
    f = pl.kernel(
        body,
        out_type=jax.ShapeDtypeStruct((NC, nrows, W128), jnp.float32),
        mesh=_MESH,
        scratch_types=[
            pltpu.VMEM((CH,), jnp.int32),
            pltpu.VMEM((CH,), jnp.int32),
            pltpu.VMEM((nch, CH), jnp.int32),
            pltpu.VMEM((CH, W128), jnp.float32),
            pltpu.VMEM((CH, W128), jnp.float32),
            pltpu.VMEM_SHARED((nrows, W128), jnp.float32),
            pltpu.SemaphoreType.DMA,
            pltpu.SemaphoreType.DMA,
            pltpu.SemaphoreType.DMA,
            pltpu.SemaphoreType.DMA,
        ],
    )
    return f(table, srcp, dstp3, zinit)


def _msg1_body(pe5, p0, p1, zp, mw, mb, out):
    z = jnp.max(zp[...])
    agg = (p0[...][:NPOS] + p1[...][:NPOS]) / z
    pen = pe5[...] + agg
    m = jnp.maximum(pen @ mw[...] + mb[...], 0.0)
    top = jnp.concatenate(
        [m, jnp.ones((NPOS, 16), jnp.float32),
         jnp.zeros((NPOS, W128 - EMB - 16), jnp.float32)], axis=1)
    out[...] = jnp.concatenate(
        [top, jnp.zeros((POSR - NPOS, W128), jnp.float32)], axis=0)


def _msg1(pe5, p0, p1, zp, mw, mb):
    return pl.pallas_call(
        _msg1_body,
        out_shape=jax.ShapeDtypeStruct((POSR, W128), jnp.float32),
    )(pe5, p0, p1, zp, mw, mb)


def _msg2_body(p0, p1, idx, ptab, gsel, esel, u1, u2, ub, tw, tb, out):
    s5 = (p0[...] + p1[...])[:NPOS]
    cnt = jnp.maximum(s5 @ esel[...], 1.0)
    agg = (s5 @ gsel[...]) / cnt
    c2 = agg @ u2[...] + ub[...]
    oh = (idx[...] == lax.broadcasted_iota(jnp.int32, (NPOS, 16), 1)
          ).astype(jnp.float32)
    pos = oh @ ptab[...]
    for _ in range(3):
        pos = jnp.maximum(pos @ u1[...] + c2, 0.0)
    m = jnp.maximum(pos[:NTEAM] @ tw[...] + tb[...], 0.0)
    top = jnp.concatenate(
        [m, jnp.ones((NTEAM, 16), jnp.float32),
         jnp.zeros((NTEAM, W128 - EMB - 16), jnp.float32)], axis=1)
    out[...] = jnp.concatenate(
        [top, jnp.zeros((TEAMR - NTEAM, W128), jnp.float32)], axis=0)


def _msg2(p0, p1, idx, ptab, gsel, esel, u1, u2, ub, tw, tb):
    return pl.pallas_call(
        _msg2_body,
        out_shape=jax.ShapeDtypeStruct((TEAMR, W128), jnp.float32),
    )(p0, p1, idx, ptab, gsel, esel, u1, u2, ub, tw, tb)


def _team_body(p0, p1, idx, ttab, gsel, esel, u1, u2, ub, out):
    s = (p0[...] + p1[...])[:NTEAM]
    cnt = jnp.maximum(s @ esel[...], 1.0)
    agg = (s @ gsel[...]) / cnt
    c2 = agg @ u2[...] + ub[...]
    oh = (idx[...] == lax.broadcasted_iota(jnp.int32, (NTEAM, 32), 1)
          ).astype(jnp.float32)
    team = oh @ ttab[...]
    for _ in range(3):
        team = jnp.maximum(team @ u1[...] + c2, 0.0)
    out[...] = team


def _team(p0, p1, idx, ttab, gsel, esel, u1, u2, ub):
    return pl.pallas_call(
        _team_body,
        out_shape=jax.ShapeDtypeStruct((NTEAM, EMB), jnp.float32),
    )(p0, p1, idx, ttab, gsel, esel, u1, u2, ub)


def _pad_edges(arr, n_pad, fill):
    return jnp.concatenate(
        [arr.astype(jnp.int32),
         jnp.full((n_pad - arr.shape[0],), fill, jnp.int32)])


@jax.jit
def kernel(player_features, position_indices, team_indices,
           player_to_position_edges, position_to_team_edges, chemistry_edges,
           enc_W1, enc_b1, enc_W2, enc_b2,
           attn_Wq, attn_bq, attn_Wk, attn_bk, attn_Wv, attn_bv,
           pos_table, team_table,
           p2p_msg_W, p2p_msg_b, p2p_upd_W, p2p_upd_b,
           p2t_msg_W, p2t_msg_b, p2t_upd_W, p2t_upd_b):
    r2 = lambda b: b.reshape(1, -1)

    pe, t1, t2 = _encode_qkv(
        player_features, enc_W1, r2(enc_b1), enc_W2, r2(enc_b2),
        attn_Wq, r2(attn_bq), attn_Wk, r2(attn_bk), attn_Wv, r2(attn_bv))

    csrc = _pad_edges(chemistry_edges[0], E_CHEM_PAD, 0)
    cdst = _pad_edges(chemistry_edges[1], E_CHEM_PAD, 0)
    cdst_cl = jnp.minimum(cdst, NPOS).reshape(NW, NCH_CHEM, CH)
    qg, kvg = _gather2(t1, t2, csrc, cdst)
    msgs, zp = _wmsg(qg, kvg)
    aggc = _scatadd(msgs, cdst_cl, POSR, E_CHEM_PAD, EMB, NCH_CHEM)

    mp = _msg1(pe[:NPOS], aggc[0], aggc[1], zp,
               p2p_msg_W, r2(p2p_msg_b))

    psrc = _pad_edges(player_to_position_edges[0], E_P2P_PAD, 0)
    pdst = _pad_edges(player_to_position_edges[1], E_P2P_PAD, NPOS
                      ).reshape(NW, NCH_CHEM, CH)
    aggp = _agg_pass(mp, psrc, pdst, POSR, E_P2P_PAD, NCH_CHEM)

    gsel = jnp.concatenate(
        [jnp.eye(EMB, dtype=jnp.float32),
         jnp.zeros((W128 - EMB, EMB), jnp.float32)], axis=0)
    esel = jnp.concatenate(
        [jnp.zeros((EMB, EMB), jnp.float32),
         jnp.full((16, EMB), 1.0 / 16.0, jnp.float32),
         jnp.zeros((W128 - EMB - 16, EMB), jnp.float32)], axis=0)

    ptab16 = jnp.concatenate(
        [pos_table, jnp.zeros((6, EMB), jnp.float32)], axis=0)
    mt = _msg2(aggp[0], aggp[1], position_indices.astype(jnp.int32)[:, None],
               ptab16, gsel, esel,
               p2p_upd_W[:EMB], p2p_upd_W[EMB:], r2(p2p_upd_b),
               p2t_msg_W, r2(p2t_msg_b))

    tsrc = _pad_edges(position_to_team_edges[0], E_P2T_PAD, 0)
    tdst = _pad_edges(position_to_team_edges[1], E_P2T_PAD, NTEAM
                      ).reshape(NW, NCH_P2T, CH)
    aggt = _agg_pass(mt, tsrc, tdst, TEAMR, E_P2T_PAD, NCH_P2T)

    return _team(aggt[0], aggt[1], team_indices.astype(jnp.int32)[:, None],
                 team_table, gsel, esel,
                 p2t_upd_W[:EMB], p2t_upd_W[EMB:], r2(p2t_upd_b))
